# baseline (device time: 184259 ns/iter reference)
import jax
import jax.numpy as jnp
from jax import lax
from jax.experimental import pallas as pl
from jax.experimental.pallas import tpu as pltpu

N_DEV = 8


def _gelu(y):
    c = 0.7978845608028654
    return 0.5 * y * (1.0 + jnp.tanh(c * (y + 0.044715 * y * y * y)))


def kernel(x, w_mat):
    m_per, k = x.shape
    _, n_per = w_mat.shape

    def body(x_ref, w_ref, out_ref, comm_ref, send_sems, recv_sems):
        my = lax.axis_index("i")
        left = lax.rem(my + N_DEV - 1, N_DEV)
        right = lax.rem(my + 1, N_DEV)

        barrier_sem = pltpu.get_barrier_semaphore()
        for nbr in (left, right):
            pl.semaphore_signal(
                barrier_sem, inc=1,
                device_id=(nbr,), device_id_type=pl.DeviceIdType.MESH,
            )
        pl.semaphore_wait(barrier_sem, 2)

        comm_ref[pl.ds(my, 1), :, :] = x_ref[:, :].reshape(1, m_per, k)
        w = w_ref[:, :]
        y = jnp.dot(x_ref[:, :], w, preferred_element_type=jnp.float32)
        out_ref[pl.ds(my * m_per, m_per), :] = _gelu(y)

        for h in range(N_DEV - 1):
            send_origin = lax.rem(my + N_DEV - h, N_DEV)
            rdma = pltpu.make_async_remote_copy(
                src_ref=comm_ref.at[send_origin],
                dst_ref=comm_ref.at[send_origin],
                send_sem=send_sems.at[h],
                recv_sem=recv_sems.at[h],
                device_id=(right,),
                device_id_type=pl.DeviceIdType.MESH,
            )
            rdma.start()
            rdma.wait()

            recv_origin = lax.rem(my + N_DEV - h - 1, N_DEV)
            chunk = comm_ref[pl.ds(recv_origin, 1), :, :].reshape(m_per, k)
            y = jnp.dot(chunk, w, preferred_element_type=jnp.float32)
            out_ref[pl.ds(recv_origin * m_per, m_per), :] = _gelu(y)

    return pl.pallas_call(
        body,
        out_shape=jax.ShapeDtypeStruct((N_DEV * m_per, n_per), jnp.float32),
        in_specs=[
            pl.BlockSpec(memory_space=pltpu.VMEM),
            pl.BlockSpec(memory_space=pltpu.VMEM),
        ],
        out_specs=pl.BlockSpec(memory_space=pltpu.VMEM),
        scratch_shapes=[
            pltpu.VMEM((N_DEV, m_per, k), x.dtype),
            pltpu.SemaphoreType.DMA((N_DEV - 1,)),
            pltpu.SemaphoreType.DMA((N_DEV - 1,)),
        ],
        compiler_params=pltpu.CompilerParams(collective_id=0),
    )(x, w_mat)


# device time: 40971 ns/iter; 4.4973x vs baseline; 4.4973x over previous
import jax
import jax.numpy as jnp
from jax import lax
from jax.experimental import pallas as pl
from jax.experimental.pallas import tpu as pltpu

N_DEV = 8
R3_SPLIT = ((0, 96), (96, 96), (192, 64))


def _gelu(y):
    c = 0.7978845608028654
    return 0.5 * y * (1.0 + jnp.tanh(c * (y + 0.044715 * y * y * y)))


def kernel(x, w_mat):
    m_per, k = x.shape
    _, n_per = w_mat.shape

    def body(x_ref, w_ref, out_ref, comm_ref, send_sems, recv_sems):
        my = lax.axis_index("i")
        nbrs = [my ^ 1, my ^ 3, my ^ 4]

        barrier_sem = pltpu.get_barrier_semaphore()
        for nbr in nbrs:
            pl.semaphore_signal(
                barrier_sem, inc=1,
                device_id=(nbr,), device_id_type=pl.DeviceIdType.MESH,
            )
        pl.semaphore_wait(barrier_sem, 3)

        w = w_ref[:, :].astype(jnp.bfloat16)
        x_bf = x_ref[:, :].astype(jnp.bfloat16)
        comm_ref[pl.ds(my, 1), :, :] = x_bf.reshape(1, m_per, k)

        def send(sem_idx, chunk_slot, dst, rows=None):
            src = comm_ref.at[chunk_slot]
            if rows is not None:
                src = comm_ref.at[chunk_slot, pl.ds(rows[0], rows[1])]
            rdma = pltpu.make_async_remote_copy(
                src_ref=src, dst_ref=src,
                send_sem=send_sems.at[sem_idx],
                recv_sem=recv_sems.at[sem_idx],
                device_id=(dst,), device_id_type=pl.DeviceIdType.MESH,
            )
            rdma.start()
            return rdma

        def gemm(slot):
            chunk = comm_ref[pl.ds(slot, 1), :, :].reshape(m_per, k)
            y = jnp.dot(chunk, w, preferred_element_type=jnp.float32)
            out_ref[pl.ds(slot * m_per, m_per), :] = _gelu(y)

        rdmas = []
        for d in range(3):
            rdmas.append(send(d, my, nbrs[d]))
        gemm(my)

        r2_chunk = [my ^ 3, my ^ 4, my ^ 1]
        r2_needs = [1, 2, 0]
        for d in range(3):
            rdmas[r2_needs[d]].wait_recv()
            rdmas.append(send(3 + d, r2_chunk[d], nbrs[d]))
        for slot in (my ^ 1, my ^ 3, my ^ 4):
            gemm(slot)

        r3_chunk = [my ^ 7, my ^ 5, my ^ 2]
        r3_needs = [4, 5, 3]
        for d in range(3):
            rdmas[r3_needs[d]].wait_recv()
            rdmas.append(send(6 + d, r3_chunk[d], nbrs[d], rows=R3_SPLIT[d]))
        for slot in (my ^ 2, my ^ 7, my ^ 5):
            gemm(slot)

        for d in range(6, 9):
            rdmas[d].wait_recv()
        gemm(my ^ 6)

        for r in rdmas:
            r.wait_send()

    return pl.pallas_call(
        body,
        out_shape=jax.ShapeDtypeStruct((N_DEV * m_per, n_per), jnp.float32),
        in_specs=[
            pl.BlockSpec(memory_space=pltpu.VMEM),
            pl.BlockSpec(memory_space=pltpu.VMEM),
        ],
        out_specs=pl.BlockSpec(memory_space=pltpu.VMEM),
        scratch_shapes=[
            pltpu.VMEM((N_DEV, m_per, k), jnp.bfloat16),
            pltpu.SemaphoreType.DMA((9,)),
            pltpu.SemaphoreType.DMA((9,)),
        ],
        compiler_params=pltpu.CompilerParams(collective_id=0),
    )(x, w_mat)


# device time: 39606 ns/iter; 4.6523x vs baseline; 1.0345x over previous
import jax
import jax.numpy as jnp
from jax import lax
from jax.experimental import pallas as pl
from jax.experimental.pallas import tpu as pltpu

N_DEV = 8
HALF = 128
R3_ROWS = ((0, 96), (160, 96), (96, 64))


def _gelu(y):
    c = 0.7978845608028654
    return 0.5 * y * (1.0 + jnp.tanh(c * (y + 0.044715 * y * y * y)))


def kernel(x, w_mat):
    m_per, k = x.shape
    _, n_per = w_mat.shape

    def body(x_ref, w_ref, out_ref, comm_ref, send_sems, recv_sems):
        my = lax.axis_index("i")
        nbrs = [my ^ 1, my ^ 3, my ^ 4]

        barrier_sem = pltpu.get_barrier_semaphore()
        for nbr in nbrs:
            pl.semaphore_signal(
                barrier_sem, inc=1,
                device_id=(nbr,), device_id_type=pl.DeviceIdType.MESH,
            )
        pl.semaphore_wait(barrier_sem, 3)

        def send(sem_idx, chunk_slot, dst, rows):
            src = comm_ref.at[chunk_slot, pl.ds(rows[0], rows[1])]
            rdma = pltpu.make_async_remote_copy(
                src_ref=src, dst_ref=src,
                send_sem=send_sems.at[sem_idx],
                recv_sem=recv_sems.at[sem_idx],
                device_id=(dst,), device_id_type=pl.DeviceIdType.MESH,
            )
            rdma.start()
            return rdma

        def gemm(slot):
            chunk = comm_ref[pl.ds(slot, 1), :, :].reshape(m_per, k)
            y = jnp.dot(chunk, w, preferred_element_type=jnp.float32)
            out_ref[pl.ds(slot * m_per, m_per), :] = _gelu(y)

        rd = {}
        for h in range(2):
            rows = (h * HALF, HALF)
            comm_ref[pl.ds(my, 1), pl.ds(h * HALF, HALF), :] = (
                x_ref[pl.ds(h * HALF, HALF), :]
                .astype(jnp.bfloat16).reshape(1, HALF, k)
            )
            for d in range(3):
                rd[("r1", d, h)] = send(d * 2 + h, my, nbrs[d], rows)

        w = w_ref[:, :].astype(jnp.bfloat16)
        gemm(my)

        r2_chunk = [my ^ 3, my ^ 4, my ^ 1]
        r2_src = [1, 2, 0]
        for h in range(2):
            for d in range(3):
                rd[("r1", r2_src[d], h)].wait_recv()
                rd[("r2", d, h)] = send(
                    6 + d * 2 + h, r2_chunk[d], nbrs[d], (h * HALF, HALF)
                )
        for slot in (my ^ 1, my ^ 3, my ^ 4):
            gemm(slot)

        r3_chunk = [my ^ 7, my ^ 5, my ^ 2]
        r3_gate = [("r2", 1, 0), ("r2", 2, 1), ("r2", 0, 1)]
        for d in range(3):
            rd[r3_gate[d]].wait_recv()
            rd[("r3", d)] = send(12 + d, r3_chunk[d], nbrs[d], R3_ROWS[d])

        for key in [("r2", 0, 0), ("r2", 1, 1), ("r2", 2, 0)]:
            rd[key].wait_recv()
        for slot in (my ^ 2, my ^ 7, my ^ 5):
            gemm(slot)

        for d in range(3):
            rd[("r3", d)].wait_recv()
        gemm(my ^ 6)

        for r in rd.values():
            r.wait_send()

    return pl.pallas_call(
        body,
        out_shape=jax.ShapeDtypeStruct((N_DEV * m_per, n_per), jnp.float32),
        in_specs=[
            pl.BlockSpec(memory_space=pltpu.VMEM),
            pl.BlockSpec(memory_space=pltpu.VMEM),
        ],
        out_specs=pl.BlockSpec(memory_space=pltpu.VMEM),
        scratch_shapes=[
            pltpu.VMEM((N_DEV, m_per, k), jnp.bfloat16),
            pltpu.SemaphoreType.DMA((15,)),
            pltpu.SemaphoreType.DMA((15,)),
        ],
        compiler_params=pltpu.CompilerParams(collective_id=0),
    )(x, w_mat)


# device time: 39249 ns/iter; 4.6946x vs baseline; 1.0091x over previous
import jax
import jax.numpy as jnp
from jax import lax
from jax.experimental import pallas as pl
from jax.experimental.pallas import tpu as pltpu

N_DEV = 8
HALF = 128
R3_ROWS = ((0, 96), (160, 96), (96, 64))


def _gelu(y):
    c = 0.7978845608028654
    return 0.5 * y * (1.0 + jnp.tanh(c * (y + 0.044715 * y * y * y)))


def kernel(x, w_mat):
    m_per, k = x.shape
    _, n_per = w_mat.shape

    def body(x_ref, w_ref, out_ref, comm_ref, send_sems, recv_sems):
        my = lax.axis_index("i")
        nbrs = [my ^ 1, my ^ 3, my ^ 4]

        barrier_sem = pltpu.get_barrier_semaphore()
        for nbr in nbrs:
            pl.semaphore_signal(
                barrier_sem, inc=1,
                device_id=(nbr,), device_id_type=pl.DeviceIdType.MESH,
            )
        pl.semaphore_wait(barrier_sem, 3)

        def send(sem_idx, chunk_slot, dst, rows):
            src = comm_ref.at[chunk_slot, pl.ds(rows[0], rows[1])]
            rdma = pltpu.make_async_remote_copy(
                src_ref=src, dst_ref=src,
                send_sem=send_sems.at[sem_idx],
                recv_sem=recv_sems.at[sem_idx],
                device_id=(dst,), device_id_type=pl.DeviceIdType.MESH,
            )
            rdma.start()
            return rdma

        def gemm(slot):
            chunk = comm_ref[pl.ds(slot, 1), :, :].reshape(m_per, k)
            y = jnp.dot(chunk, w, preferred_element_type=jnp.float32)
            out_ref[pl.ds(slot * m_per, m_per), :] = _gelu(y)

        rd = {}
        for h in range(2):
            rows = (h * HALF, HALF)
            comm_ref[pl.ds(my, 1), pl.ds(h * HALF, HALF), :] = (
                x_ref[pl.ds(h * HALF, HALF), :]
                .astype(jnp.bfloat16).reshape(1, HALF, k)
            )
            for d in range(3):
                rd[("r1", d, h)] = send(d * 2 + h, my, nbrs[d], rows)

        w = w_ref[:, :].astype(jnp.bfloat16)
        gemm(my)

        r2_chunk = [my ^ 3, my ^ 4, my ^ 1]
        r2_src = [1, 2, 0]
        for h in range(2):
            for d in range(3):
                rd[("r1", r2_src[d], h)].wait_recv()
                rd[("r2", d, h)] = send(
                    6 + d * 2 + h, r2_chunk[d], nbrs[d], (h * HALF, HALF)
                )
        for slot in (my ^ 1, my ^ 3, my ^ 4):
            gemm(slot)

        r3_chunk = [my ^ 7, my ^ 5, my ^ 2]
        r3_gate = [("r2", 1, 0), ("r2", 2, 1), ("r2", 0, 1)]
        for d in range(3):
            rd[r3_gate[d]].wait_recv()
            rd[("r3", d)] = send(12 + d, r3_chunk[d], nbrs[d], R3_ROWS[d])

        for key in [("r2", 0, 0), ("r2", 1, 1), ("r2", 2, 0)]:
            rd[key].wait_recv()
        for slot in (my ^ 2, my ^ 7, my ^ 5):
            gemm(slot)

        for d in (2, 0, 1):
            rd[("r3", d)].wait_recv()
            start, size = R3_ROWS[d]
            chunk = comm_ref[
                pl.ds(my ^ 6, 1), pl.ds(start, size), :
            ].reshape(size, k)
            y = jnp.dot(chunk, w, preferred_element_type=jnp.float32)
            out_ref[pl.ds((my ^ 6) * m_per + start, size), :] = _gelu(y)

        for r in rd.values():
            r.wait_send()

    return pl.pallas_call(
        body,
        out_shape=jax.ShapeDtypeStruct((N_DEV * m_per, n_per), jnp.float32),
        in_specs=[
            pl.BlockSpec(memory_space=pltpu.VMEM),
            pl.BlockSpec(memory_space=pltpu.VMEM),
        ],
        out_specs=pl.BlockSpec(memory_space=pltpu.VMEM),
        scratch_shapes=[
            pltpu.VMEM((N_DEV, m_per, k), jnp.bfloat16),
            pltpu.SemaphoreType.DMA((15,)),
            pltpu.SemaphoreType.DMA((15,)),
        ],
        compiler_params=pltpu.CompilerParams(collective_id=0),
    )(x, w_mat)
